# baseline (speedup 1.0000x reference)
import functools

import jax
import jax.numpy as jnp
from jax import lax
from jax.experimental import pallas as pl
from jax.experimental.pallas import tpu as pltpu
from jax.experimental.pallas import tpu_sc as plsc

D = 1024
S = 4096
B = 4
NC, NS = 2, 16
NW = NC * NS
S_PER_W = S // NW
C = 32
L = 16
NG = S_PER_W // C


def _sc_add(x, pe):
    mesh = plsc.VectorSubcoreMesh(
        core_axis_name="c", subcore_axis_name="s", num_cores=NC, num_subcores=NS
    )

    @functools.partial(
        pl.kernel,
        out_type=jax.ShapeDtypeStruct((B, S, D), jnp.float32),
        mesh=mesh,
        scratch_types=[
            pltpu.VMEM((C, D), jnp.float32),
            pltpu.VMEM((C, D), jnp.float32),
            pltpu.VMEM((C, D), jnp.float32),
            pltpu.SemaphoreType.DMA,
            pltpu.SemaphoreType.DMA,
            pltpu.SemaphoreType.DMA,
            pltpu.SemaphoreType.DMA,
            pltpu.SemaphoreType.DMA,
        ],
    )
    def k(x_hbm, pe_hbm, out_hbm, xb0, xb1, pb, sx0, sx1, so0, so1, sp):
        xb = (xb0, xb1)
        sx, so = (sx0, sx1), (so0, so1)
        cid = lax.axis_index("c")
        sid = lax.axis_index("s")
        wid = sid * NC + cid
        s_base = wid * S_PER_W

        def start_x(g, b):
            return pltpu.async_copy(
                x_hbm.at[b, pl.ds(s_base + g * C, C), :], xb[b % 2], sx[b % 2]
            )

        def start_pe(g):
            return pltpu.async_copy(pe_hbm.at[pl.ds(s_base + g * C, C), :], pb, sp)

        def wait_x0():
            pltpu.make_async_copy(x_hbm.at[0, pl.ds(0, C), :], xb[0], sx[0]).wait()

        def wait_pe():
            pltpu.make_async_copy(pe_hbm.at[pl.ds(0, C), :], pb, sp).wait()

        def wait_out(parity):
            pltpu.make_async_copy(
                xb[parity], out_hbm.at[0, pl.ds(0, C), :], so[parity]
            ).wait()

        pltpu.async_copy(xb[1], out_hbm.at[0, pl.ds(s_base, C), :], so[1])
        start_pe(0)
        start_x(0, 0)

        def group(g, _):
            for b in range(B):
                p = b % 2
                if b == 0:
                    wait_pe()
                    wait_x0()
                else:
                    x_d[0].wait()
                wait_out(1 - p)
                if b + 1 < B:
                    x_d[0] = start_x(g, b + 1)
                else:
                    g1 = jnp.minimum(g + 1, NG - 1)
                    x_d[0] = start_x(g1, 0)

                def body(i):
                    r = lax.shift_right_logical(i, 10)
                    c = pl.multiple_of(lax.bitwise_and(i, D - 1), L)
                    plsc.addupdate(xb[p].at[r, pl.ds(c, L)], pb[r, pl.ds(c, L)])

                plsc.parallel_loop(0, C * D, L, unroll=8)(body)
                if b == B - 1:
                    start_pe(jnp.minimum(g + 1, NG - 1))
                pltpu.async_copy(
                    xb[p], out_hbm.at[b, pl.ds(s_base + g * C, C), :], so[p]
                )
            return 0

        x_d = [None]
        lax.fori_loop(0, NG, group, 0)
        wait_x0()
        wait_pe()
        wait_out(1)

    return k(x, pe)


def kernel(x, pos_embedding):
    return _sc_add(x, pos_embedding)


# speedup vs baseline: 1.0397x; 1.0397x over previous
import functools

import jax
import jax.numpy as jnp
from jax import lax
from jax.experimental import pallas as pl
from jax.experimental.pallas import tpu as pltpu
from jax.experimental.pallas import tpu_sc as plsc

D = 1024
S = 4096
B = 4
NC, NS = 2, 16
NW = NC * NS
S_PER_W = S // NW
C = 32
L = 16
NG = S_PER_W // C
CH = NG * B


def _sc_add(x, pe):
    mesh = plsc.VectorSubcoreMesh(
        core_axis_name="c", subcore_axis_name="s", num_cores=NC, num_subcores=NS
    )

    @functools.partial(
        pl.kernel,
        out_type=jax.ShapeDtypeStruct((B, S, D), jnp.float32),
        mesh=mesh,
        scratch_types=[
            pltpu.VMEM((C, D), jnp.float32),
            pltpu.VMEM((C, D), jnp.float32),
            pltpu.VMEM((C, D), jnp.float32),
            pltpu.SemaphoreType.DMA,
            pltpu.SemaphoreType.DMA,
            pltpu.SemaphoreType.DMA,
            pltpu.SemaphoreType.DMA,
            pltpu.SemaphoreType.DMA,
        ],
    )
    def k(x_hbm, pe_hbm, out_hbm, xb0, xb1, pb, sx0, sx1, so0, so1, sp):
        xb = (xb0, xb1)
        sx, so = (sx0, sx1), (so0, so1)
        cid = lax.axis_index("c")
        sid = lax.axis_index("s")
        wid = sid * NC + cid
        s_base = wid * S_PER_W

        def start_x(kk):
            g, b = divmod(kk, B)
            return pltpu.async_copy(
                x_hbm.at[b, pl.ds(s_base + g * C, C), :], xb[kk % 2], sx[kk % 2]
            )

        def start_pe(g):
            return pltpu.async_copy(
                pe_hbm.at[pl.ds(s_base + g * C, C), :], pb, sp
            )

        out_d = [None, None]
        pe_d = start_pe(0)
        x_d = start_x(0)
        for kk in range(CH):
            p = kk % 2
            g, b = divmod(kk, B)
            if b == 0:
                pe_d.wait()
            x_d.wait()
            if kk + 1 < CH:
                if out_d[(kk + 1) % 2] is not None:
                    out_d[(kk + 1) % 2].wait()
                x_d = start_x(kk + 1)

            def body(i):
                r = lax.shift_right_logical(i, 10)
                c = pl.multiple_of(lax.bitwise_and(i, D - 1), L)
                plsc.addupdate(xb[p].at[r, pl.ds(c, L)], pb[r, pl.ds(c, L)])

            plsc.parallel_loop(0, C * D, L, unroll=8)(body)
            if b == B - 1 and g + 1 < NG:
                pe_d = start_pe(g + 1)
            out_d[p] = pltpu.async_copy(
                xb[p], out_hbm.at[b, pl.ds(s_base + g * C, C), :], so[p]
            )
        out_d[0].wait()
        out_d[1].wait()

    return k(x, pe)


def kernel(x, pos_embedding):
    return _sc_add(x, pos_embedding)
